# baseline (speedup 1.0000x reference)
import functools

import jax
import jax.numpy as jnp
import numpy as np
from jax import lax
from jax.experimental import pallas as pl
from jax.experimental.pallas import tpu as pltpu
from jax.experimental.pallas import tpu_sc as plsc

F = 26
FH = 13
D = 32
VOCAB = 100000
NC, NS, L = 2, 16, 16
NW = NC * NS
C = 256
G = 128

TBLK = 65536
Q8 = TBLK // 8
QSH = Q8.bit_length() - 1

V_TOTAL = F * VOCAB
NBLK = -(-V_TOTAL // TBLK)
SPLIT_ROW = FH * VOCAB
BA_END = -(-SPLIT_ROW // TBLK)
BB_START = SPLIT_ROW // TBLK


def _offs(f0):
    k = C * FH
    return (((np.arange(k, dtype=np.int32) % FH) + f0) * VOCAB).reshape(
        k // G, G)


def _tc_relayout(tab_t, b0, b1):
    def body(in_ref, out_ref):
        x = in_ref[...]
        xi = lax.bitcast_convert_type(x, jnp.int32)
        xr = xi + jnp.int32(0x8000)
        lo = xr[0:16, :]
        hi = xr[16:32, :]
        packed = jnp.bitwise_or(
            lax.shift_right_logical(lo, jnp.int32(16)),
            jnp.bitwise_and(hi, jnp.int32(-65536)),
        )
        y = jnp.concatenate(
            [packed[:, j * Q8:(j + 1) * Q8] for j in range(8)], axis=0
        )
        out_ref[...] = y.T

    return pl.pallas_call(
        body,
        grid=(b1 - b0,),
        in_specs=[pl.BlockSpec((D, TBLK), lambda i: (0, i + b0))],
        out_specs=pl.BlockSpec((Q8, 128), lambda i: (i + b0, 0)),
        out_shape=jax.ShapeDtypeStruct((NBLK * Q8, 128), jnp.int32),
    )(tab_t)


def _embedding_bag(idx2d, offs2d, table):
    K = C * FH
    NG = K // G
    B = idx2d.shape[0] * G // FH
    rows_per_w = B // NW
    n_chunks = rows_per_w // C
    idxrows_per_chunk = K // G

    mesh = plsc.VectorSubcoreMesh(core_axis_name="c", subcore_axis_name="s")

    @functools.partial(
        pl.kernel,
        out_type=jax.ShapeDtypeStruct((B, D), jnp.float32),
        mesh=mesh,
        compiler_params=pltpu.CompilerParams(
            use_tc_tiling_on_sc=False, needs_layout_passes=False),
        scratch_types=[
            pltpu.VMEM((NG, G), jnp.int32),
            pltpu.VMEM((NG, G), jnp.int32),
            pltpu.VMEM((K, 16), jnp.int32),
            pltpu.VMEM((C, D), jnp.float32),
            pltpu.SemaphoreType.DMA,
        ],
    )
    def body(idx_hbm, offs_hbm, tab_hbm, out_hbm, idx_v, offs_v, rows_v, out_v, sem):
        wid = lax.axis_index("s") * NC + lax.axis_index("c")
        bag_base = wid * rows_per_w
        pltpu.sync_copy(offs_hbm, offs_v)

        def chunk(j, carry):
            cbag = bag_base + j * C
            crow = cbag * FH // G
            pltpu.sync_copy(idx_hbm.at[pl.ds(crow, idxrows_per_chunk)], idx_v)

            def offs(i, _):
                for g in range(NG):
                    sl = pl.ds(i * L, L)
                    r = idx_v[g, sl] + offs_v[g, sl]
                    band = jnp.bitwise_and(r, TBLK - 1)
                    s = (r - band) + jnp.left_shift(
                        jnp.bitwise_and(r, Q8 - 1), 3
                    ) + jnp.right_shift(band, QSH)
                    idx_v[g, sl] = s
                return 0

            lax.fori_loop(0, G // L, offs, 0)

            copies = [
                pltpu.async_copy(
                    tab_hbm.at[idx_v.at[g]], rows_v.at[pl.ds(g * G, G)], sem
                )
                for g in range(NG)
            ]
            for cp in copies:
                cp.wait()

            hi_mask = jnp.int32(-65536)
            sh16 = jnp.int32(16)

            def red(c, _):
                rbase = c * FH
                v = rows_v[rbase, :]
                acc_lo = plsc.bitcast(jnp.left_shift(v, sh16), jnp.float32)
                acc_hi = plsc.bitcast(jnp.bitwise_and(v, hi_mask), jnp.float32)
                for f in range(1, FH):
                    v = rows_v[rbase + f, :]
                    acc_lo = acc_lo + plsc.bitcast(
                        jnp.left_shift(v, sh16), jnp.float32)
                    acc_hi = acc_hi + plsc.bitcast(
                        jnp.bitwise_and(v, hi_mask), jnp.float32)
                out_v[c, pl.ds(0, L)] = acc_lo
                out_v[c, pl.ds(L, L)] = acc_hi
                return 0

            lax.fori_loop(0, C, red, 0)
            pltpu.sync_copy(out_v, out_hbm.at[pl.ds(cbag, C)])
            return carry

        lax.fori_loop(0, n_chunks, chunk, 0)

    return body(idx2d, offs2d, table)


def kernel(input_, embed_weight):
    B = input_.shape[0]
    tab_t = embed_weight.T
    idx_a = input_[:, :FH].reshape(B * FH // G, G)
    idx_b = input_[:, FH:].reshape(B * FH // G, G)
    tab_a = _tc_relayout(tab_t, 0, BA_END).reshape(-1, 16)
    out_a = _embedding_bag(idx_a, jnp.asarray(_offs(0)), tab_a)
    tab_b = _tc_relayout(tab_t, BB_START, NBLK).reshape(-1, 16)
    out_b = _embedding_bag(idx_b, jnp.asarray(_offs(FH)), tab_b)
    return out_a + out_b


# speedup vs baseline: 1.0866x; 1.0866x over previous
import functools

import jax
import jax.numpy as jnp
import numpy as np
from jax import lax
from jax.experimental import pallas as pl
from jax.experimental.pallas import tpu as pltpu
from jax.experimental.pallas import tpu_sc as plsc

F = 26
D = 32
VOCAB = 100000
NC, NS, L = 2, 16, 16
NW = NC * NS
C = 256
K = C * F
G = 128
NG = K // G

TBLK = 131072
Q8 = TBLK // 8
QSH = Q8.bit_length() - 1

_OFFS = ((np.arange(K, dtype=np.int32) % F) * VOCAB).reshape(NG, G)


def _tc_relayout(tab_t):
    V = tab_t.shape[1]
    n_blk = pl.cdiv(V, TBLK)

    def body(in_ref, out_ref):
        x = in_ref[...]
        xi = lax.bitcast_convert_type(x, jnp.int32)
        xr = xi + jnp.int32(0x8000)
        lo = xr[0:16, :]
        hi = xr[16:32, :]
        packed = jnp.bitwise_or(
            lax.shift_right_logical(lo, jnp.int32(16)),
            jnp.bitwise_and(hi, jnp.int32(-65536)),
        )
        y = jnp.concatenate(
            [packed[:, j * Q8:(j + 1) * Q8] for j in range(8)], axis=0
        )
        out_ref[...] = y.T

    return pl.pallas_call(
        body,
        grid=(n_blk,),
        in_specs=[pl.BlockSpec((D, TBLK), lambda i: (0, i))],
        out_specs=pl.BlockSpec((Q8, 128), lambda i: (i, 0)),
        out_shape=jax.ShapeDtypeStruct((n_blk * Q8, 128), jnp.int32),
    )(tab_t)


def _embedding_bag(idx2d, offs2d, table):
    B = idx2d.shape[0] * G // F
    rows_per_w = B // NW
    n_chunks = rows_per_w // C
    idxrows_per_chunk = K // G

    mesh = plsc.VectorSubcoreMesh(core_axis_name="c", subcore_axis_name="s")

    @functools.partial(
        pl.kernel,
        out_type=jax.ShapeDtypeStruct((B, D), jnp.float32),
        mesh=mesh,
        compiler_params=pltpu.CompilerParams(
            use_tc_tiling_on_sc=False, needs_layout_passes=False),
        scratch_types=[
            pltpu.VMEM((NG, G), jnp.int32),
            pltpu.VMEM((NG, G), jnp.int32),
            pltpu.VMEM((K, 16), jnp.int32),
            pltpu.VMEM((C, D), jnp.float32),
            pltpu.SemaphoreType.DMA,
        ],
    )
    def body(idx_hbm, offs_hbm, tab_hbm, out_hbm, idx_v, offs_v, rows_v, out_v, sem):
        wid = lax.axis_index("s") * NC + lax.axis_index("c")
        bag_base = wid * rows_per_w
        pltpu.sync_copy(offs_hbm, offs_v)

        def chunk(j, carry):
            cbag = bag_base + j * C
            crow = cbag * F // G
            pltpu.sync_copy(idx_hbm.at[pl.ds(crow, idxrows_per_chunk)], idx_v)

            def offs(i, _):
                for g in range(NG):
                    sl = pl.ds(i * L, L)
                    r = idx_v[g, sl] + offs_v[g, sl]
                    band = jnp.bitwise_and(r, TBLK - 1)
                    s = (r - band) + jnp.left_shift(
                        jnp.bitwise_and(r, Q8 - 1), 3
                    ) + jnp.right_shift(band, QSH)
                    idx_v[g, sl] = s
                return 0

            lax.fori_loop(0, G // L, offs, 0)

            copies = [
                pltpu.async_copy(
                    tab_hbm.at[idx_v.at[g]], rows_v.at[pl.ds(g * G, G)], sem
                )
                for g in range(NG)
            ]
            for cp in copies:
                cp.wait()

            hi_mask = jnp.int32(-65536)
            sh16 = jnp.int32(16)

            def red(c, _):
                rbase = c * F
                v = rows_v[rbase, :]
                acc_lo = plsc.bitcast(jnp.left_shift(v, sh16), jnp.float32)
                acc_hi = plsc.bitcast(jnp.bitwise_and(v, hi_mask), jnp.float32)
                for f in range(1, F):
                    v = rows_v[rbase + f, :]
                    acc_lo = acc_lo + plsc.bitcast(
                        jnp.left_shift(v, sh16), jnp.float32)
                    acc_hi = acc_hi + plsc.bitcast(
                        jnp.bitwise_and(v, hi_mask), jnp.float32)
                out_v[c, pl.ds(0, L)] = acc_lo
                out_v[c, pl.ds(L, L)] = acc_hi
                return 0

            lax.fori_loop(0, C, red, 0)
            pltpu.sync_copy(out_v, out_hbm.at[pl.ds(cbag, C)])
            return carry

        lax.fori_loop(0, n_chunks, chunk, 0)

    return body(idx2d, offs2d, table)


def kernel(input_, embed_weight):
    B = input_.shape[0]
    idx2d = input_.reshape(B * F // G, G)
    tab_t = embed_weight.T
    tab128 = _tc_relayout(tab_t)
    tab_lin = tab128.reshape(-1, 16)
    return _embedding_bag(idx2d, jnp.asarray(_OFFS), tab_lin)
